# Initial kernel scaffold; baseline (speedup 1.0000x reference)
import jax
import jax.numpy as jnp
from jax.experimental import pallas as pl


def kernel(x, expert_w, expert_b, router_w, router_b):
    raise NotImplementedError("write your pallas kernel here")


# speedup vs baseline: 1.4732x; 1.4732x over previous
import functools

import jax
import jax.numpy as jnp
from jax.experimental import pallas as pl
from jax.experimental.pallas import tpu as pltpu

B = 256
C = 96
P = 16
E = 8
TAPS = 9
K_COL = TAPS * C
N_OUT = E * C
TB = 8
S = P * P


def _router_body(xf_ref, rw_ref, rb_ref, gate_ref):
    logits = jnp.dot(xf_ref[...], rw_ref[...],
                     preferred_element_type=jnp.float32)
    logits = logits + rb_ref[0:1, :]
    m = jnp.max(logits, axis=-1, keepdims=True)
    ex = jnp.exp(logits - m)
    gate_ref[...] = ex / jnp.sum(ex, axis=-1, keepdims=True)


def _moe_body(xp_ref, wf_ref, b_ref, g_ref, r_ref, out_ref, xcol_ref):
    for dj in range(3):
        xs = xp_ref[:, :, dj:dj + P, :]
        for di in range(3):
            k = di * 3 + dj
            xcol_ref[:, k * C:(k + 1) * C] = (
                xs[:, di:di + P].reshape(TB * S, C))
    acc = jnp.dot(xcol_ref[...], wf_ref[...],
                  preferred_element_type=jnp.float32)
    y = acc + b_ref[0:1, :]
    y = y.reshape(TB, S, N_OUT) * g_ref[...][:, None, :]
    y = jnp.maximum(y, 0.0).reshape(TB * S, N_OUT)
    out = jnp.dot(y.astype(jnp.bfloat16), r_ref[...],
                  preferred_element_type=jnp.float32)
    out_ref[...] = out.reshape(TB, P, P, C)


@jax.jit
def kernel(x, expert_w, expert_b, router_w, router_b):
    xf = x.reshape(B, C * P * P)
    rb_tile = jnp.tile(router_b[None, :], (8, 1))
    gate = pl.pallas_call(
        _router_body,
        grid=(2,),
        in_specs=[
            pl.BlockSpec((B // 2, C * P * P), lambda i: (i, 0)),
            pl.BlockSpec((C * P * P, E), lambda i: (0, 0)),
            pl.BlockSpec((8, E), lambda i: (0, 0)),
        ],
        out_specs=pl.BlockSpec((B // 2, E), lambda i: (i, 0)),
        out_shape=jax.ShapeDtypeStruct((B, E), jnp.float32),
        compiler_params=pltpu.CompilerParams(
            dimension_semantics=("parallel",)),
    )(xf, router_w, rb_tile)
    gate_big = jnp.repeat(gate, C, axis=1)

    xb = x.astype(jnp.bfloat16).transpose(0, 2, 3, 1)
    xp = jnp.pad(xb, ((0, 0), (1, 1), (1, 1), (0, 0)))
    wf = expert_w.transpose(3, 4, 2, 0, 1).reshape(K_COL, N_OUT)
    wf = wf.astype(jnp.bfloat16)
    b_tile = jnp.tile(expert_b.reshape(1, N_OUT), (8, 1))
    rmat = jnp.tile(jnp.eye(C, dtype=jnp.bfloat16), (E, 1))

    out = pl.pallas_call(
        _moe_body,
        grid=(B // TB,),
        in_specs=[
            pl.BlockSpec((TB, P + 2, P + 2, C), lambda i: (i, 0, 0, 0)),
            pl.BlockSpec((K_COL, N_OUT), lambda i: (0, 0)),
            pl.BlockSpec((8, N_OUT), lambda i: (0, 0)),
            pl.BlockSpec((TB, N_OUT), lambda i: (i, 0)),
            pl.BlockSpec((N_OUT, C), lambda i: (0, 0)),
        ],
        out_specs=pl.BlockSpec((TB, P, P, C), lambda i: (i, 0, 0, 0)),
        out_shape=jax.ShapeDtypeStruct((B, P, P, C), jnp.float32),
        scratch_shapes=[pltpu.VMEM((TB * S, K_COL), jnp.bfloat16)],
        compiler_params=pltpu.CompilerParams(
            dimension_semantics=("parallel",)),
    )(xp, wf, b_tile, gate_big, rmat)

    return out.transpose(0, 3, 1, 2)


# speedup vs baseline: 1.9947x; 1.3540x over previous
import jax
import jax.numpy as jnp
from jax import lax
from jax.experimental import pallas as pl
from jax.experimental.pallas import tpu as pltpu

B = 256
C = 96
P = 16
E = 8
S = P * P
TAPS = 9
K_IM = TAPS * C
K_PAD = 872
N_OUT = E * C
TB = 8


def _router_body(xf_ref, rw_ref, rb_ref, gate_ref):
    logits = jnp.dot(xf_ref[...], rw_ref[...],
                     preferred_element_type=jnp.float32)
    logits = logits + rb_ref[0:1, :]
    m = jnp.max(logits, axis=-1, keepdims=True)
    ex = jnp.exp(logits - m)
    gate_ref[...] = ex / jnp.sum(ex, axis=-1, keepdims=True)


def _moe_body(x_ref, wf_ref, g_ref, out_ref, xcol_ref):
    lane = lax.broadcasted_iota(jnp.int32, (1, S), 1)
    row = lane // P
    col = lane % P
    pad_iota = lax.broadcasted_iota(jnp.int32, (K_PAD - K_IM, TB * S), 0)
    xcol_ref[K_IM:K_PAD, :] = jnp.where(pad_iota == 0, 1.0, 0.0
                                        ).astype(jnp.bfloat16)
    for b in range(TB):
        xb = x_ref[b].astype(jnp.bfloat16)
        for di in range(3):
            for dj in range(3):
                k = di * 3 + dj
                off = (di - 1) * P + (dj - 1)
                sh = pltpu.roll(xb, (-off) % S, 1) if off else xb
                ok_i = ((row + (di - 1) >= 0) & (row + (di - 1) < P))
                ok_j = ((col + (dj - 1) >= 0) & (col + (dj - 1) < P))
                xcol_ref[k * C:(k + 1) * C, b * S:(b + 1) * S] = (
                    jnp.where(ok_i & ok_j, sh, jnp.bfloat16(0.0)))
    acc = jnp.dot(wf_ref[...], xcol_ref[...],
                  preferred_element_type=jnp.float32)
    for b in range(TB):
        o = g_ref[b, 0] * jnp.maximum(acc[0:C, b * S:(b + 1) * S], 0.0)
        for e in range(1, E):
            o = o + g_ref[b, e] * jnp.maximum(
                acc[e * C:(e + 1) * C, b * S:(b + 1) * S], 0.0)
        out_ref[b] = o


@jax.jit
def kernel(x, expert_w, expert_b, router_w, router_b):
    xf = x.reshape(B, C * S)
    rb_tile = jnp.tile(router_b[None, :], (8, 1))
    gate = pl.pallas_call(
        _router_body,
        grid=(2,),
        in_specs=[
            pl.BlockSpec((B // 2, C * S), lambda i: (i, 0)),
            pl.BlockSpec((C * S, E), lambda i: (0, 0)),
            pl.BlockSpec((8, E), lambda i: (0, 0)),
        ],
        out_specs=pl.BlockSpec((B // 2, E), lambda i: (i, 0)),
        out_shape=jax.ShapeDtypeStruct((B, E), jnp.float32),
        compiler_params=pltpu.CompilerParams(
            dimension_semantics=("parallel",)),
    )(xf, router_w, rb_tile)

    wf = expert_w.transpose(0, 1, 3, 4, 2).reshape(N_OUT, K_IM)
    wf = jnp.concatenate(
        [wf, expert_b.reshape(N_OUT, 1),
         jnp.zeros((N_OUT, K_PAD - K_IM - 1), jnp.float32)], axis=1)
    wf = wf.astype(jnp.bfloat16)

    xr = x.reshape(B, C, S)
    out = pl.pallas_call(
        _moe_body,
        grid=(B // TB,),
        in_specs=[
            pl.BlockSpec((TB, C, S), lambda i: (i, 0, 0)),
            pl.BlockSpec((N_OUT, K_PAD), lambda i: (0, 0)),
            pl.BlockSpec((TB, E), lambda i: (i, 0),
                         memory_space=pltpu.SMEM),
        ],
        out_specs=pl.BlockSpec((TB, C, S), lambda i: (i, 0, 0)),
        out_shape=jax.ShapeDtypeStruct((B, C, S), jnp.float32),
        scratch_shapes=[pltpu.VMEM((K_PAD, TB * S), jnp.bfloat16)],
        compiler_params=pltpu.CompilerParams(
            dimension_semantics=("parallel",)),
    )(xr, wf, gate)

    return out.reshape(B, C, P, P)


# speedup vs baseline: 2.0968x; 1.0512x over previous
import jax
import jax.numpy as jnp
from jax import lax
from jax.experimental import pallas as pl
from jax.experimental.pallas import tpu as pltpu

B = 256
C = 96
P = 16
E = 8
S = P * P
TAPS = 9
K_IM = TAPS * C
K_PAD = 872
N_OUT = E * C
TB = 8


def _router_body(xf_ref, rw_ref, rb_ref, gate_ref):
    logits = jnp.dot(xf_ref[...], rw_ref[...],
                     preferred_element_type=jnp.float32)
    logits = logits + rb_ref[0:1, :]
    m = jnp.max(logits, axis=-1, keepdims=True)
    ex = jnp.exp(logits - m)
    gate_ref[...] = ex / jnp.sum(ex, axis=-1, keepdims=True)


def _moe_body(x_ref, wf_ref, g_ref, out_ref, xcol_ref):
    lane = lax.broadcasted_iota(jnp.int32, (1, S), 1)
    row = lane // P
    col = lane % P
    pad_iota = lax.broadcasted_iota(jnp.int32, (K_PAD - K_IM, TB * S), 0)
    xcol_ref[K_IM:K_PAD, :] = jnp.where(pad_iota == 0, 1.0, 0.0
                                        ).astype(jnp.bfloat16)
    for b in range(TB):
        xb = x_ref[b]
        for di in range(3):
            for dj in range(3):
                k = di * 3 + dj
                off = (di - 1) * P + (dj - 1)
                sh = pltpu.roll(xb, (-off) % S, 1) if off else xb
                ok_i = ((row + (di - 1) >= 0) & (row + (di - 1) < P))
                ok_j = ((col + (dj - 1) >= 0) & (col + (dj - 1) < P))
                xcol_ref[k * C:(k + 1) * C, b * S:(b + 1) * S] = (
                    jnp.where(ok_i & ok_j, sh, jnp.bfloat16(0.0)))
    acc = jnp.dot(wf_ref[...], xcol_ref[...],
                  preferred_element_type=jnp.float32)
    for b in range(TB):
        g0 = g_ref[b, 0].astype(jnp.bfloat16)
        o = g0 * jnp.maximum(
            acc[0:C, b * S:(b + 1) * S], 0.0).astype(jnp.bfloat16)
        for e in range(1, E):
            ge = g_ref[b, e].astype(jnp.bfloat16)
            o = o + ge * jnp.maximum(
                acc[e * C:(e + 1) * C, b * S:(b + 1) * S],
                0.0).astype(jnp.bfloat16)
        out_ref[b] = o


@jax.jit
def kernel(x, expert_w, expert_b, router_w, router_b):
    xb16 = x.astype(jnp.bfloat16)
    xm = xb16.reshape(B, C * S)
    xr = xb16.reshape(B, C, S)

    rb_tile = jnp.tile(router_b[None, :], (8, 1))
    gate = pl.pallas_call(
        _router_body,
        grid=(2,),
        in_specs=[
            pl.BlockSpec((B // 2, C * S), lambda i: (i, 0)),
            pl.BlockSpec((C * S, E), lambda i: (0, 0)),
            pl.BlockSpec((8, E), lambda i: (0, 0)),
        ],
        out_specs=pl.BlockSpec((B // 2, E), lambda i: (i, 0)),
        out_shape=jax.ShapeDtypeStruct((B, E), jnp.float32),
        compiler_params=pltpu.CompilerParams(
            dimension_semantics=("parallel",)),
    )(xm, router_w.astype(jnp.bfloat16), rb_tile)

    wf = expert_w.transpose(0, 1, 3, 4, 2).reshape(N_OUT, K_IM)
    wf = jnp.concatenate(
        [wf, expert_b.reshape(N_OUT, 1),
         jnp.zeros((N_OUT, K_PAD - K_IM - 1), jnp.float32)], axis=1)
    wf = wf.astype(jnp.bfloat16)

    out = pl.pallas_call(
        _moe_body,
        grid=(B // TB,),
        in_specs=[
            pl.BlockSpec((TB, C, S), lambda i: (i, 0, 0)),
            pl.BlockSpec((N_OUT, K_PAD), lambda i: (0, 0)),
            pl.BlockSpec((TB, E), lambda i: (i, 0),
                         memory_space=pltpu.SMEM),
        ],
        out_specs=pl.BlockSpec((TB, C, S), lambda i: (i, 0, 0)),
        out_shape=jax.ShapeDtypeStruct((B, C, S), jnp.bfloat16),
        scratch_shapes=[pltpu.VMEM((K_PAD, TB * S), jnp.bfloat16)],
        compiler_params=pltpu.CompilerParams(
            dimension_semantics=("parallel",)),
    )(xr, wf, gate)

    return out.astype(jnp.float32).reshape(B, C, P, P)


# speedup vs baseline: 2.1613x; 1.0308x over previous
import jax
import jax.numpy as jnp
from jax import lax
from jax.experimental import pallas as pl
from jax.experimental.pallas import tpu as pltpu

B = 256
C = 96
P = 16
E = 8
S = P * P
TAPS = 9
K_IM = TAPS * C
K_PAD = 872
N_OUT = E * C
TB = 16


def _router_body(xf_ref, rw_ref, rb_ref, gate_ref):
    logits = jnp.dot(xf_ref[...], rw_ref[...],
                     preferred_element_type=jnp.float32)
    logits = logits + rb_ref[0:1, :]
    m = jnp.max(logits, axis=-1, keepdims=True)
    ex = jnp.exp(logits - m)
    gate_ref[...] = ex / jnp.sum(ex, axis=-1, keepdims=True)


def _moe_body(x_ref, wf_ref, g_ref, out_ref, xcol_ref):
    lane = lax.broadcasted_iota(jnp.int32, (1, S), 1)
    row = lane // P
    col = lane % P
    pad_iota = lax.broadcasted_iota(jnp.int32, (K_PAD - K_IM, TB * S), 0)
    xcol_ref[K_IM:K_PAD, :] = jnp.where(pad_iota == 0, 1.0, 0.0
                                        ).astype(jnp.bfloat16)
    for b in range(TB):
        xb = x_ref[b]
        for di in range(3):
            for dj in range(3):
                k = di * 3 + dj
                off = (di - 1) * P + (dj - 1)
                sh = pltpu.roll(xb, (-off) % S, 1) if off else xb
                ok_i = ((row + (di - 1) >= 0) & (row + (di - 1) < P))
                ok_j = ((col + (dj - 1) >= 0) & (col + (dj - 1) < P))
                xcol_ref[k * C:(k + 1) * C, b * S:(b + 1) * S] = (
                    jnp.where(ok_i & ok_j, sh, jnp.bfloat16(0.0)))
    acc = jnp.dot(wf_ref[...], xcol_ref[...],
                  preferred_element_type=jnp.float32)
    for b in range(TB):
        g0 = g_ref[b, 0].astype(jnp.bfloat16)
        o = g0 * jnp.maximum(
            acc[0:C, b * S:(b + 1) * S], 0.0).astype(jnp.bfloat16)
        for e in range(1, E):
            ge = g_ref[b, e].astype(jnp.bfloat16)
            o = o + ge * jnp.maximum(
                acc[e * C:(e + 1) * C, b * S:(b + 1) * S],
                0.0).astype(jnp.bfloat16)
        out_ref[b] = o


@jax.jit
def kernel(x, expert_w, expert_b, router_w, router_b):
    xb16 = x.astype(jnp.bfloat16)
    xm = xb16.reshape(B, C * S)
    xr = xb16.reshape(B, C, S)

    rb_tile = jnp.tile(router_b[None, :], (8, 1))
    gate = pl.pallas_call(
        _router_body,
        grid=(2,),
        in_specs=[
            pl.BlockSpec((B // 2, C * S), lambda i: (i, 0)),
            pl.BlockSpec((C * S, E), lambda i: (0, 0)),
            pl.BlockSpec((8, E), lambda i: (0, 0)),
        ],
        out_specs=pl.BlockSpec((B // 2, E), lambda i: (i, 0)),
        out_shape=jax.ShapeDtypeStruct((B, E), jnp.float32),
        compiler_params=pltpu.CompilerParams(
            dimension_semantics=("parallel",)),
    )(xm, router_w.astype(jnp.bfloat16), rb_tile)

    wf = expert_w.transpose(0, 1, 3, 4, 2).reshape(N_OUT, K_IM)
    wf = jnp.concatenate(
        [wf, expert_b.reshape(N_OUT, 1),
         jnp.zeros((N_OUT, K_PAD - K_IM - 1), jnp.float32)], axis=1)
    wf = wf.astype(jnp.bfloat16)

    out = pl.pallas_call(
        _moe_body,
        grid=(B // TB,),
        in_specs=[
            pl.BlockSpec((TB, C, S), lambda i: (i, 0, 0)),
            pl.BlockSpec((N_OUT, K_PAD), lambda i: (0, 0)),
            pl.BlockSpec((TB, E), lambda i: (i, 0),
                         memory_space=pltpu.SMEM),
        ],
        out_specs=pl.BlockSpec((TB, C, S), lambda i: (i, 0, 0)),
        out_shape=jax.ShapeDtypeStruct((B, C, S), jnp.bfloat16),
        scratch_shapes=[pltpu.VMEM((K_PAD, TB * S), jnp.bfloat16)],
        compiler_params=pltpu.CompilerParams(
            dimension_semantics=("parallel",)),
    )(xr, wf, gate)

    return out.astype(jnp.float32).reshape(B, C, P, P)


# speedup vs baseline: 2.3742x; 1.0985x over previous
import jax
import jax.numpy as jnp
from jax import lax
from jax.experimental import pallas as pl
from jax.experimental.pallas import tpu as pltpu

B = 256
C = 96
P = 16
E = 8
S = P * P
TAPS = 9
K_IM = TAPS * C
K_PAD = 872
N_OUT = E * C
TB = 16


def _router_body(xf_ref, rw_ref, rb_ref, gate_ref):
    logits = jnp.dot(xf_ref[...].astype(jnp.bfloat16), rw_ref[...],
                     preferred_element_type=jnp.float32)
    logits = logits + rb_ref[0:1, :]
    m = jnp.max(logits, axis=-1, keepdims=True)
    ex = jnp.exp(logits - m)
    gate_ref[...] = ex / jnp.sum(ex, axis=-1, keepdims=True)


GP = 2
NG = TB // GP


def _moe_body(x_ref, wf_ref, g_ref, out_ref, xcol0, xcol1):
    lane = lax.broadcasted_iota(jnp.int32, (1, S), 1)
    row = lane // P
    col = lane % P
    pad_iota = lax.broadcasted_iota(jnp.int32, (K_PAD - K_IM, GP * S), 0)
    pad_rows = jnp.where(pad_iota == 0, 1.0, 0.0).astype(jnp.bfloat16)
    bufs = (xcol0, xcol1)

    def build(g):
        xc = bufs[g % 2]
        xc[K_IM:K_PAD, :] = pad_rows
        for i in range(GP):
            xb = x_ref[g * GP + i].reshape(C, S).astype(jnp.bfloat16)
            for di in range(3):
                for dj in range(3):
                    k = di * 3 + dj
                    off = (di - 1) * P + (dj - 1)
                    sh = pltpu.roll(xb, (-off) % S, 1) if off else xb
                    ok_i = ((row + (di - 1) >= 0) & (row + (di - 1) < P))
                    ok_j = ((col + (dj - 1) >= 0) & (col + (dj - 1) < P))
                    xc[k * C:(k + 1) * C, i * S:(i + 1) * S] = (
                        jnp.where(ok_i & ok_j, sh, jnp.bfloat16(0.0)))

    def mix(g, acc):
        for i in range(GP):
            b = g * GP + i
            o = g_ref[b, 0] * jnp.maximum(acc[0:C, i * S:(i + 1) * S], 0.0)
            for e in range(1, E):
                o = o + g_ref[b, e] * jnp.maximum(
                    acc[e * C:(e + 1) * C, i * S:(i + 1) * S], 0.0)
            out_ref[b] = o.astype(jnp.bfloat16)

    build(0)
    for g in range(NG):
        acc = jnp.dot(wf_ref[...], bufs[g % 2][...],
                      preferred_element_type=jnp.float32)
        if g + 1 < NG:
            build(g + 1)
        mix(g, acc)


@jax.jit
def kernel(x, expert_w, expert_b, router_w, router_b):
    xm = x.reshape(B, C * S)

    rb_tile = jnp.tile(router_b[None, :], (8, 1))
    gate = pl.pallas_call(
        _router_body,
        grid=(2,),
        in_specs=[
            pl.BlockSpec((B // 2, C * S), lambda i: (i, 0)),
            pl.BlockSpec((C * S, E), lambda i: (0, 0)),
            pl.BlockSpec((8, E), lambda i: (0, 0)),
        ],
        out_specs=pl.BlockSpec((B // 2, E), lambda i: (i, 0)),
        out_shape=jax.ShapeDtypeStruct((B, E), jnp.float32),
        compiler_params=pltpu.CompilerParams(
            dimension_semantics=("parallel",)),
    )(xm, router_w.astype(jnp.bfloat16), rb_tile)

    wf = expert_w.transpose(0, 1, 3, 4, 2).reshape(N_OUT, K_IM)
    wf = jnp.concatenate(
        [wf, expert_b.reshape(N_OUT, 1),
         jnp.zeros((N_OUT, K_PAD - K_IM - 1), jnp.float32)], axis=1)
    wf = wf.astype(jnp.bfloat16)

    out = pl.pallas_call(
        _moe_body,
        grid=(B // TB,),
        in_specs=[
            pl.BlockSpec((TB, C * S), lambda i: (i, 0)),
            pl.BlockSpec((N_OUT, K_PAD), lambda i: (0, 0)),
            pl.BlockSpec((TB, E), lambda i: (i, 0),
                         memory_space=pltpu.SMEM),
        ],
        out_specs=pl.BlockSpec((TB, C, S), lambda i: (i, 0, 0)),
        out_shape=jax.ShapeDtypeStruct((B, C, S), jnp.bfloat16),
        scratch_shapes=[pltpu.VMEM((K_PAD, GP * S), jnp.bfloat16),
                        pltpu.VMEM((K_PAD, GP * S), jnp.bfloat16)],
        compiler_params=pltpu.CompilerParams(
            dimension_semantics=("parallel",)),
    )(xm, wf, gate)

    return out.astype(jnp.float32).reshape(B, C, P, P)
